# baseline (speedup 1.0000x reference)
import math
import functools

import jax
import jax.numpy as jnp
from jax import lax
from jax.experimental import pallas as pl
from jax.experimental.pallas import tpu as pltpu
from jax.experimental.pallas import tpu_sc as plsc

SIZE_ = 32000
PAD_ = 0
SMOOTH_ = 0.1
CONF_ = 1.0 - SMOOTH_
SVAL_ = SMOOTH_ / (SIZE_ - 2)
K_ = (SIZE_ - 2) * SVAL_ * math.log(SVAL_) + CONF_ * math.log(CONF_)

B_ = 2048
BC_ = 1280

_NC = 2
_NS = 16
_NW = _NC * _NS
_BPW = B_ // _NW
_L = 16


def _rowsum_body(out_ref, t_ref, acc_ref):
    j = pl.program_id(0)
    rs = jnp.sum(out_ref[...], axis=1, keepdims=True)
    masked = jnp.where(t_ref[...] != PAD_, rs, 0.0)
    partial = jnp.sum(masked)

    @pl.when(j == 0)
    def _init():
        acc_ref[0, 0] = 0.0

    acc_ref[0, 0] = acc_ref[0, 0] + partial


def _sc_body(flat_hbm, tgt_hbm, out_hbm, tgt_v, idx_v, idx0_v, g_v, g0_v,
             part_v, sem):
    wid = lax.axis_index("s") * _NC + lax.axis_index("c")
    base = wid * _BPW
    pltpu.sync_copy(tgt_hbm.at[pl.ds(base, _BPW)], tgt_v)
    for k in range(_BPW // _L):
        t = tgt_v[pl.ds(k * _L, _L)]
        rows = base + k * _L + lax.iota(jnp.int32, _L)
        idx_v[pl.ds(k * _L, _L)] = rows * SIZE_ + t
        idx0_v[pl.ds(k * _L, _L)] = rows * SIZE_
    pltpu.async_copy(flat_hbm.at[idx_v], g_v, sem).wait()
    pltpu.async_copy(flat_hbm.at[idx0_v], g0_v, sem).wait()
    acc = jnp.zeros((_L,), jnp.float32)
    for k in range(_BPW // _L):
        t = tgt_v[pl.ds(k * _L, _L)]
        g = g_v[pl.ds(k * _L, _L)]
        g0 = g0_v[pl.ds(k * _L, _L)]
        val = K_ + SVAL_ * g0 - (CONF_ - SVAL_) * g
        acc = acc + jnp.where(t != PAD_, val, 0.0)
    part_v[...] = acc
    pltpu.sync_copy(part_v, out_hbm.at[wid])


_sc_gather = functools.partial(
    pl.kernel,
    mesh=plsc.VectorSubcoreMesh(core_axis_name="c", subcore_axis_name="s"),
    out_type=jax.ShapeDtypeStruct((_NW, _L), jnp.float32),
    scratch_types=[
        pltpu.VMEM((_BPW,), jnp.int32),
        pltpu.VMEM((_BPW,), jnp.int32),
        pltpu.VMEM((_BPW,), jnp.int32),
        pltpu.VMEM((_BPW,), jnp.float32),
        pltpu.VMEM((_BPW,), jnp.float32),
        pltpu.VMEM((_L,), jnp.float32),
        pltpu.SemaphoreType.DMA,
    ],
)(_sc_body)


@jax.jit
def kernel(output, target, one_hot):
    del one_hot
    t32 = target.astype(jnp.int32)
    sc_parts = _sc_gather(output.reshape(B_ * SIZE_), t32)

    t2 = t32.reshape(B_, 1)
    acc = pl.pallas_call(
        _rowsum_body,
        grid=(SIZE_ // BC_,),
        in_specs=[
            pl.BlockSpec((B_, BC_), lambda j: (0, j)),
            pl.BlockSpec((B_, 1), lambda j: (0, 0)),
        ],
        out_specs=pl.BlockSpec(
            (1, 1), lambda j: (0, 0), memory_space=pltpu.SMEM
        ),
        out_shape=jax.ShapeDtypeStruct((1, 1), jnp.float32),
    )(output, t2)
    return jnp.sum(sc_parts) - SVAL_ * acc[0, 0]


# speedup vs baseline: 2.8967x; 2.8967x over previous
import math
import functools

import jax
import jax.numpy as jnp
from jax import lax
from jax.experimental import pallas as pl
from jax.experimental.pallas import tpu as pltpu

SIZE_ = 32000
PAD_ = 0
SMOOTH_ = 0.1
CONF_ = 1.0 - SMOOTH_
SVAL_ = SMOOTH_ / (SIZE_ - 2)
K_ = (SIZE_ - 2) * SVAL_ * math.log(SVAL_) + CONF_ * math.log(CONF_)

B_ = 2048
BC_ = 1280


def _loss_body(out_ref, t_ref, acc_ref):
    j = pl.program_id(0)
    out_blk = out_ref[...]
    t_blk = t_ref[...]
    nonpad = t_blk != PAD_

    col0 = j * BC_
    colids = col0 + lax.broadcasted_iota(jnp.int32, (B_, BC_), 1)
    w = jnp.where(colids == t_blk, CONF_, SVAL_)
    w = jnp.where(colids == 0, 0.0, w)
    w = jnp.where(nonpad, w, 0.0)
    partial = jnp.sum(out_blk * w)

    @pl.when(j == 0)
    def _init():
        cnt = jnp.sum(nonpad.astype(jnp.float32))
        acc_ref[0, 0] = K_ * cnt

    acc_ref[0, 0] = acc_ref[0, 0] - partial


@jax.jit
def kernel(output, target, one_hot):
    del one_hot
    t2 = target.astype(jnp.int32).reshape(B_, 1)
    acc = pl.pallas_call(
        _loss_body,
        grid=(SIZE_ // BC_,),
        in_specs=[
            pl.BlockSpec((B_, BC_), lambda j: (0, j)),
            pl.BlockSpec((B_, 1), lambda j: (0, 0)),
        ],
        out_specs=pl.BlockSpec(
            (1, 1), lambda j: (0, 0), memory_space=pltpu.SMEM
        ),
        out_shape=jax.ShapeDtypeStruct((1, 1), jnp.float32),
    )(output, t2)
    return acc[0, 0]
